# baseline (speedup 1.0000x reference)
import functools

import jax
import jax.numpy as jnp
from jax import lax
from jax.experimental import pallas as pl
from jax.experimental.pallas import tpu as pltpu

NN = 50000
EE = 800000
RB = 2000
GRID = NN // RB


def _rowspec(w):
    return pl.BlockSpec((RB, w), lambda i: (i, 0))


def _fullspec(shape):
    nd = len(shape)
    return pl.BlockSpec(shape, lambda i: (0,) * nd)


def _tc_pre_body(v_ref, n_ref, idx_ref, w_ref):
    v = v_ref[...]
    nrm = n_ref[...]
    idx_parts = []
    w_parts = []
    for pt in (v, v + nrm, v - nrm):
        gd = jnp.clip(pt[:, 0:1], 0.0, 63.0)
        gh = jnp.clip(pt[:, 1:2], 0.0, 63.0)
        gw = jnp.clip(pt[:, 2:3], 0.0, 63.0)
        d0 = jnp.floor(gd)
        h0 = jnp.floor(gh)
        w0 = jnp.floor(gw)
        wd = gd - d0
        wh = gh - h0
        ww = gw - w0
        d0i = d0.astype(jnp.int32)
        h0i = h0.astype(jnp.int32)
        w0i = w0.astype(jnp.int32)
        d1i = jnp.minimum(d0i + 1, 63)
        h1i = jnp.minimum(h0i + 1, 63)
        w1i = jnp.minimum(w0i + 1, 63)
        for di, wdp in ((d0i, 1.0 - wd), (d1i, wd)):
            for hi, whp in ((h0i, 1.0 - wh), (h1i, wh)):
                for wi, wwp in ((w0i, 1.0 - ww), (w1i, ww)):
                    idx_parts.append((di * 64 + hi) * 64 + wi)
                    w_parts.append(wdp * whp * wwp)
    idx_ref[...] = jnp.concatenate(idx_parts, axis=-1)
    w_ref[...] = jnp.concatenate(w_parts, axis=-1)


def _tc_pre(vertices, vertex_normals):
    return pl.pallas_call(
        _tc_pre_body,
        grid=(GRID,),
        in_specs=[_rowspec(3), _rowspec(3)],
        out_specs=[_rowspec(24), _rowspec(24)],
        out_shape=[
            jax.ShapeDtypeStruct((NN, 24), jnp.int32),
            jax.ShapeDtypeStruct((NN, 24), jnp.float32),
        ],
    )(vertices, vertex_normals)


def _tc1_body(a0_ref, a1_ref, g_ref, w24_ref, v_ref, vp_ref,
              w1rel_ref, w1root_ref, b1_ref, wlin_ref, blin_ref,
              w2root_ref, w2rel_ref,
              invc_ref, res_ref, xr2_ref, y2_ref):
    s1 = a0_ref[...] + a1_ref[...]
    cnt = s1[:, 6:7]
    invc = 1.0 / jnp.maximum(cnt, 1.0)
    invc_ref[...] = invc
    mean1 = s1[:, 0:6] * invc
    inpf = jnp.concatenate([v_ref[...], vp_ref[...]], axis=-1)
    feats = (jnp.dot(mean1, w1rel_ref[...], preferred_element_type=jnp.float32)
             + jnp.dot(inpf, w1root_ref[...], preferred_element_type=jnp.float32)
             + b1_ref[...])
    g = g_ref[...]
    w24 = w24_ref[...]
    aggs = []
    for p in range(3):
        acc = jnp.zeros((g.shape[0], 8), jnp.float32)
        for k in range(8):
            j = p * 8 + k
            acc = acc + w24[:, j:j + 1] * g[:, j * 8:(j + 1) * 8]
        aggs.append(acc)
    x = jnp.concatenate([feats] + aggs + [inpf], axis=-1)
    res_ref[...] = jnp.dot(x, wlin_ref[...], preferred_element_type=jnp.float32) + blin_ref[...]
    xr2_ref[...] = jnp.dot(x, w2root_ref[...], preferred_element_type=jnp.float32)
    y2_ref[...] = jnp.dot(x, w2rel_ref[...], preferred_element_type=jnp.float32)


def _tc1(a0, a1, g192, w24, vertices, vp, w1rel_t, w1root_t, b1r, wlin_t,
         blinr, w2root_t, w2rel_t):
    return pl.pallas_call(
        _tc1_body,
        grid=(GRID,),
        in_specs=[
            _rowspec(8), _rowspec(8), _rowspec(192), _rowspec(24),
            _rowspec(3), _rowspec(3),
            _fullspec((6, 64)), _fullspec((6, 64)), _fullspec((1, 64)),
            _fullspec((94, 64)), _fullspec((1, 64)),
            _fullspec((94, 64)), _fullspec((94, 64)),
        ],
        out_specs=[_rowspec(1), _rowspec(64), _rowspec(64), _rowspec(64)],
        out_shape=[
            jax.ShapeDtypeStruct((NN, 1), jnp.float32),
            jax.ShapeDtypeStruct((NN, 64), jnp.float32),
            jax.ShapeDtypeStruct((NN, 64), jnp.float32),
            jax.ShapeDtypeStruct((NN, 64), jnp.float32),
        ],
    )(a0, a1, g192, w24, vertices, vp, w1rel_t, w1root_t, b1r, wlin_t,
      blinr, w2root_t, w2rel_t)


def _layer_norm_relu(h, g, b):
    mu = jnp.mean(h, axis=-1, keepdims=True)
    var = jnp.mean((h - mu) * (h - mu), axis=-1, keepdims=True)
    hn = (h - mu) * jax.lax.rsqrt(var + 1e-5) * g + b
    return jnp.maximum(hn, 0.0)


def _tc2_body(s2_ref, invc_ref, xr2_ref, g2_ref, be2_ref,
              w3rel_ref, w3root_ref, y3_ref, hr3_ref):
    h = s2_ref[...] * invc_ref[...] + xr2_ref[...]
    h = _layer_norm_relu(h, g2_ref[...], be2_ref[...])
    y3_ref[...] = jnp.dot(h, w3rel_ref[...], preferred_element_type=jnp.float32)
    hr3_ref[...] = jnp.dot(h, w3root_ref[...], preferred_element_type=jnp.float32)


def _tc2(s2, invc, xr2, g2r, be2r, w3rel_t, w3root_t):
    return pl.pallas_call(
        _tc2_body,
        grid=(GRID,),
        in_specs=[
            _rowspec(64), _rowspec(1), _rowspec(64),
            _fullspec((1, 64)), _fullspec((1, 64)),
            _fullspec((64, 64)), _fullspec((64, 64)),
        ],
        out_specs=[_rowspec(64), _rowspec(64)],
        out_shape=[
            jax.ShapeDtypeStruct((NN, 64), jnp.float32),
            jax.ShapeDtypeStruct((NN, 64), jnp.float32),
        ],
    )(s2, invc, xr2, g2r, be2r, w3rel_t, w3root_t)


def _tc3_body(s3_ref, invc_ref, hr3_ref, g3_ref, be3_ref, res_ref,
              w4rel_ref, w4root_ref, y4_ref, f4r_ref):
    h = s3_ref[...] * invc_ref[...] + hr3_ref[...]
    h = _layer_norm_relu(h, g3_ref[...], be3_ref[...])
    feats = jnp.maximum(h + res_ref[...], 0.0)
    y4 = jnp.dot(feats, w4rel_ref[...], preferred_element_type=jnp.float32)
    y4_ref[...] = jnp.concatenate(
        [y4, jnp.zeros((y4.shape[0], 5), jnp.float32)], axis=-1)
    f4r_ref[...] = jnp.dot(feats, w4root_ref[...], preferred_element_type=jnp.float32)


def _tc3(s3, invc, hr3, g3r, be3r, res, w4rel_t, w4root_t):
    return pl.pallas_call(
        _tc3_body,
        grid=(GRID,),
        in_specs=[
            _rowspec(64), _rowspec(1), _rowspec(64),
            _fullspec((1, 64)), _fullspec((1, 64)), _rowspec(64),
            _fullspec((64, 3)), _fullspec((64, 3)),
        ],
        out_specs=[_rowspec(8), _rowspec(3)],
        out_shape=[
            jax.ShapeDtypeStruct((NN, 8), jnp.float32),
            jax.ShapeDtypeStruct((NN, 3), jnp.float32),
        ],
    )(s3, invc, hr3, g3r, be3r, res, w4rel_t, w4root_t)


def _tc4_body(a0_ref, a1_ref, invc_ref, f4r_ref, out_ref):
    s4 = a0_ref[...] + a1_ref[...]
    out_ref[...] = s4[:, 0:3] * invc_ref[...] + f4r_ref[...]


def _tc4(a0, a1, invc, f4r):
    return pl.pallas_call(
        _tc4_body,
        grid=(GRID,),
        in_specs=[_rowspec(8), _rowspec(8), _rowspec(1), _rowspec(3)],
        out_specs=_rowspec(3),
        out_shape=jax.ShapeDtypeStruct((NN, 3), jnp.float32),
    )(a0, a1, invc, f4r)


def _seg_sum8_halves(y8, src, dst):
    s = jax.ops.segment_sum(y8[src], dst, num_segments=NN)
    return s, jnp.zeros_like(s)


def _seg_sum64(y64, src, dst):
    return jax.ops.segment_sum(y64[src], dst, num_segments=NN)


def _grid_gather(table, idx24):
    return table[idx24.reshape(-1)].reshape(NN, 192)


def kernel(vertices, edges, vertex_normals, voxel_features, vector_potential,
           w1_root, w1_rel, b1, w2_root, w2_rel, g2, be2, w3_root, w3_rel,
           g3, be3, w_lin, b_lin, w4_root, w4_rel):
    f32 = jnp.float32
    src = jnp.asarray(edges[:, 0], jnp.int32)
    dst = jnp.asarray(edges[:, 1], jnp.int32)

    table = jnp.transpose(voxel_features, (1, 2, 3, 0)).reshape(64 * 64 * 64, 8)

    idx24, w24 = _tc_pre(vertices, vertex_normals)
    g192 = _grid_gather(table, idx24)

    ones = jnp.ones((NN, 1), f32)
    zeros = jnp.zeros((NN, 1), f32)
    y1 = jnp.concatenate([vertices, vector_potential, ones, zeros], axis=-1)
    a0, a1 = _seg_sum8_halves(y1, src, dst)

    invc, res, xr2, y2 = _tc1(
        a0, a1, g192, w24, vertices, vector_potential,
        w1_rel.T, w1_root.T, b1.reshape(1, 64), w_lin.T, b_lin.reshape(1, 64),
        w2_root.T, w2_rel.T)

    s2 = _seg_sum64(y2, src, dst)
    y3, hr3 = _tc2(s2, invc, xr2, g2.reshape(1, 64), be2.reshape(1, 64),
                   w3_rel.T, w3_root.T)
    s3 = _seg_sum64(y3, src, dst)
    y4, f4r = _tc3(s3, invc, hr3, g3.reshape(1, 64), be3.reshape(1, 64),
                   res, w4_rel.T, w4_root.T)
    b0, b1h = _seg_sum8_halves(y4, src, dst)
    return _tc4(b0, b1h, invc, f4r)


# speedup vs baseline: 3.3389x; 3.3389x over previous
import functools

import jax
import jax.numpy as jnp
from jax import lax
from jax.experimental import pallas as pl
from jax.experimental.pallas import tpu as pltpu
from jax.experimental.pallas import tpu_sc as plsc

NN = 50000
EE = 800000
RB = 2000
GRID = NN // RB

CH = 128
EP = 802816
ECH = EP // CH
ACC_R = 51200
NGF = NN * 24
NGP = 1200128
_MESH = plsc.VectorSubcoreMesh(core_axis_name="c", subcore_axis_name="s")


def _rowspec(w):
    return pl.BlockSpec((RB, w), lambda i: (i, 0))


def _fullspec(shape):
    nd = len(shape)
    return pl.BlockSpec(shape, lambda i: (0,) * nd)


def _tc_pre_body(v_ref, n_ref, idx_ref, w_ref):
    v = v_ref[...]
    nrm = n_ref[...]
    idx_parts = []
    w_parts = []
    for pt in (v, v + nrm, v - nrm):
        gd = jnp.clip(pt[:, 0:1], 0.0, 63.0)
        gh = jnp.clip(pt[:, 1:2], 0.0, 63.0)
        gw = jnp.clip(pt[:, 2:3], 0.0, 63.0)
        d0 = jnp.floor(gd)
        h0 = jnp.floor(gh)
        w0 = jnp.floor(gw)
        wd = gd - d0
        wh = gh - h0
        ww = gw - w0
        d0i = d0.astype(jnp.int32)
        h0i = h0.astype(jnp.int32)
        w0i = w0.astype(jnp.int32)
        d1i = jnp.minimum(d0i + 1, 63)
        h1i = jnp.minimum(h0i + 1, 63)
        w1i = jnp.minimum(w0i + 1, 63)
        for di, wdp in ((d0i, 1.0 - wd), (d1i, wd)):
            for hi, whp in ((h0i, 1.0 - wh), (h1i, wh)):
                for wi, wwp in ((w0i, 1.0 - ww), (w1i, ww)):
                    idx_parts.append((di * 64 + hi) * 64 + wi)
                    w_parts.append(wdp * whp * wwp)
    idx_ref[...] = jnp.concatenate(idx_parts, axis=-1)
    w_ref[...] = jnp.concatenate(w_parts, axis=-1)


def _tc_pre(vertices, vertex_normals):
    return pl.pallas_call(
        _tc_pre_body,
        grid=(GRID,),
        in_specs=[_rowspec(3), _rowspec(3)],
        out_specs=[_rowspec(24), _rowspec(24)],
        out_shape=[
            jax.ShapeDtypeStruct((NN, 24), jnp.int32),
            jax.ShapeDtypeStruct((NN, 24), jnp.float32),
        ],
    )(vertices, vertex_normals)


def _tc1_body(a0_ref, a1_ref, g_ref, w24_ref, v_ref, vp_ref,
              w1rel_ref, w1root_ref, b1_ref, wlin_ref, blin_ref,
              w2root_ref, w2rel_ref,
              invc_ref, res_ref, xr2_ref, y2_ref):
    s1 = a0_ref[...] + a1_ref[...]
    cnt = s1[:, 6:7]
    invc = 1.0 / jnp.maximum(cnt, 1.0)
    invc_ref[...] = invc
    mean1 = s1[:, 0:6] * invc
    inpf = jnp.concatenate([v_ref[...], vp_ref[...]], axis=-1)
    feats = (jnp.dot(mean1, w1rel_ref[...], preferred_element_type=jnp.float32)
             + jnp.dot(inpf, w1root_ref[...], preferred_element_type=jnp.float32)
             + b1_ref[...])
    g = g_ref[...]
    w24 = w24_ref[...]
    aggs = []
    for p in range(3):
        acc = jnp.zeros((g.shape[0], 8), jnp.float32)
        for k in range(8):
            j = p * 8 + k
            acc = acc + w24[:, j:j + 1] * g[:, j * 8:(j + 1) * 8]
        aggs.append(acc)
    x = jnp.concatenate([feats] + aggs + [inpf], axis=-1)
    res_ref[...] = jnp.dot(x, wlin_ref[...], preferred_element_type=jnp.float32) + blin_ref[...]
    xr2_ref[...] = jnp.dot(x, w2root_ref[...], preferred_element_type=jnp.float32)
    y2_ref[...] = jnp.dot(x, w2rel_ref[...], preferred_element_type=jnp.float32)


def _tc1(a0, a1, g192, w24, vertices, vp, w1rel_t, w1root_t, b1r, wlin_t,
         blinr, w2root_t, w2rel_t):
    return pl.pallas_call(
        _tc1_body,
        grid=(GRID,),
        in_specs=[
            _rowspec(8), _rowspec(8), _rowspec(192), _rowspec(24),
            _rowspec(3), _rowspec(3),
            _fullspec((6, 64)), _fullspec((6, 64)), _fullspec((1, 64)),
            _fullspec((94, 64)), _fullspec((1, 64)),
            _fullspec((94, 64)), _fullspec((94, 64)),
        ],
        out_specs=[_rowspec(1), _rowspec(64), _rowspec(64), _rowspec(64)],
        out_shape=[
            jax.ShapeDtypeStruct((NN, 1), jnp.float32),
            jax.ShapeDtypeStruct((NN, 64), jnp.float32),
            jax.ShapeDtypeStruct((NN, 64), jnp.float32),
            jax.ShapeDtypeStruct((NN, 64), jnp.float32),
        ],
    )(a0, a1, g192, w24, vertices, vp, w1rel_t, w1root_t, b1r, wlin_t,
      blinr, w2root_t, w2rel_t)


def _layer_norm_relu(h, g, b):
    mu = jnp.mean(h, axis=-1, keepdims=True)
    var = jnp.mean((h - mu) * (h - mu), axis=-1, keepdims=True)
    hn = (h - mu) * jax.lax.rsqrt(var + 1e-5) * g + b
    return jnp.maximum(hn, 0.0)


def _tc2_body(s2_ref, invc_ref, xr2_ref, g2_ref, be2_ref,
              w3rel_ref, w3root_ref, y3_ref, hr3_ref):
    h = s2_ref[...] * invc_ref[...] + xr2_ref[...]
    h = _layer_norm_relu(h, g2_ref[...], be2_ref[...])
    y3_ref[...] = jnp.dot(h, w3rel_ref[...], preferred_element_type=jnp.float32)
    hr3_ref[...] = jnp.dot(h, w3root_ref[...], preferred_element_type=jnp.float32)


def _tc2(s2, invc, xr2, g2r, be2r, w3rel_t, w3root_t):
    return pl.pallas_call(
        _tc2_body,
        grid=(GRID,),
        in_specs=[
            _rowspec(64), _rowspec(1), _rowspec(64),
            _fullspec((1, 64)), _fullspec((1, 64)),
            _fullspec((64, 64)), _fullspec((64, 64)),
        ],
        out_specs=[_rowspec(64), _rowspec(64)],
        out_shape=[
            jax.ShapeDtypeStruct((NN, 64), jnp.float32),
            jax.ShapeDtypeStruct((NN, 64), jnp.float32),
        ],
    )(s2, invc, xr2, g2r, be2r, w3rel_t, w3root_t)


def _tc3_body(s3_ref, invc_ref, hr3_ref, g3_ref, be3_ref, res_ref,
              w4rel_ref, w4root_ref, y4_ref, f4r_ref):
    h = s3_ref[...] * invc_ref[...] + hr3_ref[...]
    h = _layer_norm_relu(h, g3_ref[...], be3_ref[...])
    feats = jnp.maximum(h + res_ref[...], 0.0)
    y4 = jnp.dot(feats, w4rel_ref[...], preferred_element_type=jnp.float32)
    y4_ref[...] = jnp.concatenate(
        [y4, jnp.zeros((y4.shape[0], 5), jnp.float32)], axis=-1)
    f4r_ref[...] = jnp.dot(feats, w4root_ref[...], preferred_element_type=jnp.float32)


def _tc3(s3, invc, hr3, g3r, be3r, res, w4rel_t, w4root_t):
    return pl.pallas_call(
        _tc3_body,
        grid=(GRID,),
        in_specs=[
            _rowspec(64), _rowspec(1), _rowspec(64),
            _fullspec((1, 64)), _fullspec((1, 64)), _rowspec(64),
            _fullspec((64, 3)), _fullspec((64, 3)),
        ],
        out_specs=[_rowspec(8), _rowspec(3)],
        out_shape=[
            jax.ShapeDtypeStruct((NN, 8), jnp.float32),
            jax.ShapeDtypeStruct((NN, 3), jnp.float32),
        ],
    )(s3, invc, hr3, g3r, be3r, res, w4rel_t, w4root_t)


def _tc4_body(a0_ref, a1_ref, invc_ref, f4r_ref, out_ref):
    s4 = a0_ref[...] + a1_ref[...]
    out_ref[...] = s4[:, 0:3] * invc_ref[...] + f4r_ref[...]


def _tc4(a0, a1, invc, f4r):
    return pl.pallas_call(
        _tc4_body,
        grid=(GRID,),
        in_specs=[_rowspec(8), _rowspec(8), _rowspec(1), _rowspec(3)],
        out_specs=_rowspec(3),
        out_shape=jax.ShapeDtypeStruct((NN, 3), jnp.float32),
    )(a0, a1, invc, f4r)


def _make_seg_pass(w, split_edges):
    chunks = ECH // 32 if split_edges else ECH // 16

    @functools.partial(
        pl.kernel, mesh=_MESH,
        compiler_params=pltpu.CompilerParams(use_tc_tiling_on_sc=False),
        out_type=jax.ShapeDtypeStruct((2 * ACC_R, w), jnp.float32),
        scratch_types=[
            pltpu.VMEM((CH,), jnp.int32),
            pltpu.VMEM((CH,), jnp.int32),
            pltpu.VMEM((CH, w), jnp.float32),
            pltpu.VMEM_SHARED((ACC_R, w), jnp.float32),
            pltpu.SemaphoreType.DMA,
        ],
    )
    def seg(ytab_h, src_h, dst_h, zeros_h, out_h, sidx, didx, grows, acc, sem):
        c = lax.axis_index("c")
        s = lax.axis_index("s")
        rows_per_tile = ACC_R // 16
        def zinit(j, carry):
            r = s * rows_per_tile + j * CH
            pltpu.sync_copy(zeros_h.at[pl.ds(j * CH, CH)], acc.at[pl.ds(r, CH)])
            return carry
        lax.fori_loop(0, rows_per_tile // CH, zinit, 0)
        plsc.subcore_barrier()

        def body(k, carry):
            if split_edges:
                g = c * (ECH // 2) + k * 16 + s
            else:
                g = k * 16 + s
            base = g * CH
            pltpu.sync_copy(src_h.at[pl.ds(base, CH)], sidx)
            pltpu.sync_copy(dst_h.at[pl.ds(base, CH)], didx)
            if not split_edges:
                off = c * NN
                for j in range(CH // 16):
                    sl = pl.ds(j * 16, 16)
                    sidx[sl] = sidx[sl] + off
            pltpu.async_copy(ytab_h.at[sidx], grows, sem).wait()
            pltpu.sync_copy(grows, acc.at[didx], add=True)
            return carry
        lax.fori_loop(0, chunks, body, 0)
        plsc.subcore_barrier()

        def wout(j, carry):
            r = s * rows_per_tile + j * CH
            pltpu.sync_copy(acc.at[pl.ds(r, CH)],
                            out_h.at[pl.ds(c * ACC_R + r, CH)])
            return carry
        lax.fori_loop(0, rows_per_tile // CH, wout, 0)

    return seg


_seg8 = _make_seg_pass(8, True)
_seg32 = _make_seg_pass(32, False)


def _seg_sum8_halves(y8, src_p, dst_p, zeros8):
    out = _seg8(y8, src_p, dst_p, zeros8)
    return out[:NN], out[ACC_R:ACC_R + NN]


def _seg_sum64(y64, src_p, dst_p, zeros32):
    ytab = jnp.concatenate([y64[:, :32], y64[:, 32:]], axis=0)
    out = _seg32(ytab, src_p, dst_p, zeros32)
    return jnp.concatenate([out[:NN], out[ACC_R:ACC_R + NN]], axis=1)


@functools.partial(
    pl.kernel, mesh=_MESH,
    compiler_params=pltpu.CompilerParams(use_tc_tiling_on_sc=False),
    out_type=jax.ShapeDtypeStruct((NGP, 8), jnp.float32),
    scratch_types=[
        pltpu.VMEM((CH,), jnp.int32),
        pltpu.VMEM((CH, 8), jnp.float32),
        pltpu.SemaphoreType.DMA,
    ],
)
def _sc_grid(table_h, idx_h, out_h, idx_v, rows_v, sem):
    wid = lax.axis_index("s") * 2 + lax.axis_index("c")

    def body(k, carry):
        base = (k * 32 + wid) * CH
        pltpu.sync_copy(idx_h.at[pl.ds(base, CH)], idx_v)
        pltpu.async_copy(table_h.at[idx_v], rows_v, sem).wait()
        pltpu.sync_copy(rows_v, out_h.at[pl.ds(base, CH)])
        return carry
    lax.fori_loop(0, NGP // CH // 32, body, 0)
    return None


def _grid_gather(table, idx24):
    idx_pad = jnp.concatenate(
        [idx24.reshape(-1), jnp.zeros((NGP - NGF,), jnp.int32)])
    out = _sc_grid(table, idx_pad)
    return out[:NGF].reshape(NN, 192)


def kernel(vertices, edges, vertex_normals, voxel_features, vector_potential,
           w1_root, w1_rel, b1, w2_root, w2_rel, g2, be2, w3_root, w3_rel,
           g3, be3, w_lin, b_lin, w4_root, w4_rel):
    f32 = jnp.float32
    src = jnp.asarray(edges[:, 0], jnp.int32)
    dst = jnp.asarray(edges[:, 1], jnp.int32)
    src_p = jnp.concatenate([src, jnp.zeros((EP - EE,), jnp.int32)])
    dst_p = jnp.concatenate([dst, jnp.full((EP - EE,), NN, jnp.int32)])
    zeros8 = jnp.zeros((ACC_R // 16, 8), f32)
    zeros32 = jnp.zeros((ACC_R // 16, 32), f32)

    table = jnp.transpose(voxel_features, (1, 2, 3, 0)).reshape(64 * 64 * 64, 8)

    idx24, w24 = _tc_pre(vertices, vertex_normals)
    g192 = _grid_gather(table, idx24)

    ones = jnp.ones((NN, 1), f32)
    zeros = jnp.zeros((NN, 1), f32)
    y1 = jnp.concatenate([vertices, vector_potential, ones, zeros], axis=-1)
    a0, a1 = _seg_sum8_halves(y1, src_p, dst_p, zeros8)

    invc, res, xr2, y2 = _tc1(
        a0, a1, g192, w24, vertices, vector_potential,
        w1_rel.T, w1_root.T, b1.reshape(1, 64), w_lin.T, b_lin.reshape(1, 64),
        w2_root.T, w2_rel.T)

    s2 = _seg_sum64(y2, src_p, dst_p, zeros32)
    y3, hr3 = _tc2(s2, invc, xr2, g2.reshape(1, 64), be2.reshape(1, 64),
                   w3_rel.T, w3_root.T)
    s3 = _seg_sum64(y3, src_p, dst_p, zeros32)
    y4, f4r = _tc3(s3, invc, hr3, g3.reshape(1, 64), be3.reshape(1, 64),
                   res, w4_rel.T, w4_root.T)
    b0, b1h = _seg_sum8_halves(y4, src_p, dst_p, zeros8)
    return _tc4(b0, b1h, invc, f4r)


# speedup vs baseline: 4.2515x; 1.2733x over previous
import functools

import jax
import jax.numpy as jnp
from jax import lax
from jax.experimental import pallas as pl
from jax.experimental.pallas import tpu as pltpu
from jax.experimental.pallas import tpu_sc as plsc

NN = 50000
EE = 800000
RB = 2000
GRID = NN // RB

CH = 128
SS = 2
EP = 819200
ECH = EP // CH
ACC_R = 50176
NGF = NN * 24
NGP = 1212416
_MESH = plsc.VectorSubcoreMesh(core_axis_name="c", subcore_axis_name="s")


def _rowspec(w):
    return pl.BlockSpec((RB, w), lambda i: (i, 0))


def _fullspec(shape):
    nd = len(shape)
    return pl.BlockSpec(shape, lambda i: (0,) * nd)


def _tc_pre_body(v_ref, n_ref, idx_ref, w_ref):
    v = v_ref[...]
    nrm = n_ref[...]
    idx_parts = []
    w_parts = []
    for pt in (v, v + nrm, v - nrm):
        gd = jnp.clip(pt[:, 0:1], 0.0, 63.0)
        gh = jnp.clip(pt[:, 1:2], 0.0, 63.0)
        gw = jnp.clip(pt[:, 2:3], 0.0, 63.0)
        d0 = jnp.floor(gd)
        h0 = jnp.floor(gh)
        w0 = jnp.floor(gw)
        wd = gd - d0
        wh = gh - h0
        ww = gw - w0
        d0i = d0.astype(jnp.int32)
        h0i = h0.astype(jnp.int32)
        w0i = w0.astype(jnp.int32)
        d1i = jnp.minimum(d0i + 1, 63)
        h1i = jnp.minimum(h0i + 1, 63)
        w1i = jnp.minimum(w0i + 1, 63)
        for di, wdp in ((d0i, 1.0 - wd), (d1i, wd)):
            for hi, whp in ((h0i, 1.0 - wh), (h1i, wh)):
                for wi, wwp in ((w0i, 1.0 - ww), (w1i, ww)):
                    idx_parts.append((di * 64 + hi) * 64 + wi)
                    w_parts.append(wdp * whp * wwp)
    idx_ref[...] = jnp.concatenate(idx_parts, axis=-1)
    w_ref[...] = jnp.concatenate(w_parts, axis=-1)


def _tc_pre(vertices, vertex_normals):
    return pl.pallas_call(
        _tc_pre_body,
        grid=(GRID,),
        in_specs=[_rowspec(3), _rowspec(3)],
        out_specs=[_rowspec(24), _rowspec(24)],
        out_shape=[
            jax.ShapeDtypeStruct((NN, 24), jnp.int32),
            jax.ShapeDtypeStruct((NN, 24), jnp.float32),
        ],
    )(vertices, vertex_normals)


def _tc1_body(a0_ref, a1_ref, g_ref, w24_ref, v_ref, vp_ref,
              w1rel_ref, w1root_ref, b1_ref, wlin_ref, blin_ref,
              w2root_ref, w2rel_ref,
              invc_ref, res_ref, xr2_ref, y2_ref):
    s1 = a0_ref[...] + a1_ref[...]
    cnt = s1[:, 6:7]
    invc = 1.0 / jnp.maximum(cnt, 1.0)
    invc_ref[...] = invc
    mean1 = s1[:, 0:6] * invc
    inpf = jnp.concatenate([v_ref[...], vp_ref[...]], axis=-1)
    feats = (jnp.dot(mean1, w1rel_ref[...], preferred_element_type=jnp.float32)
             + jnp.dot(inpf, w1root_ref[...], preferred_element_type=jnp.float32)
             + b1_ref[...])
    g = g_ref[...]
    w24 = w24_ref[...]
    aggs = []
    for p in range(3):
        acc = jnp.zeros((g.shape[0], 8), jnp.float32)
        for k in range(8):
            j = p * 8 + k
            acc = acc + w24[:, j:j + 1] * g[:, j * 8:(j + 1) * 8]
        aggs.append(acc)
    x = jnp.concatenate([feats] + aggs + [inpf], axis=-1)
    res_ref[...] = jnp.dot(x, wlin_ref[...], preferred_element_type=jnp.float32) + blin_ref[...]
    xr2_ref[...] = jnp.dot(x, w2root_ref[...], preferred_element_type=jnp.float32)
    y2_ref[...] = jnp.dot(x, w2rel_ref[...], preferred_element_type=jnp.float32)


def _tc1(a0, a1, g192, w24, vertices, vp, w1rel_t, w1root_t, b1r, wlin_t,
         blinr, w2root_t, w2rel_t):
    return pl.pallas_call(
        _tc1_body,
        grid=(GRID,),
        in_specs=[
            _rowspec(8), _rowspec(8), _rowspec(192), _rowspec(24),
            _rowspec(3), _rowspec(3),
            _fullspec((6, 64)), _fullspec((6, 64)), _fullspec((1, 64)),
            _fullspec((94, 64)), _fullspec((1, 64)),
            _fullspec((94, 64)), _fullspec((94, 64)),
        ],
        out_specs=[_rowspec(1), _rowspec(64), _rowspec(64), _rowspec(64)],
        out_shape=[
            jax.ShapeDtypeStruct((NN, 1), jnp.float32),
            jax.ShapeDtypeStruct((NN, 64), jnp.float32),
            jax.ShapeDtypeStruct((NN, 64), jnp.float32),
            jax.ShapeDtypeStruct((NN, 64), jnp.float32),
        ],
    )(a0, a1, g192, w24, vertices, vp, w1rel_t, w1root_t, b1r, wlin_t,
      blinr, w2root_t, w2rel_t)


def _layer_norm_relu(h, g, b):
    mu = jnp.mean(h, axis=-1, keepdims=True)
    var = jnp.mean((h - mu) * (h - mu), axis=-1, keepdims=True)
    hn = (h - mu) * jax.lax.rsqrt(var + 1e-5) * g + b
    return jnp.maximum(hn, 0.0)


def _tc2_body(s2_ref, invc_ref, xr2_ref, g2_ref, be2_ref,
              w3rel_ref, w3root_ref, y3_ref, hr3_ref):
    h = s2_ref[...] * invc_ref[...] + xr2_ref[...]
    h = _layer_norm_relu(h, g2_ref[...], be2_ref[...])
    y3_ref[...] = jnp.dot(h, w3rel_ref[...], preferred_element_type=jnp.float32)
    hr3_ref[...] = jnp.dot(h, w3root_ref[...], preferred_element_type=jnp.float32)


def _tc2(s2, invc, xr2, g2r, be2r, w3rel_t, w3root_t):
    return pl.pallas_call(
        _tc2_body,
        grid=(GRID,),
        in_specs=[
            _rowspec(64), _rowspec(1), _rowspec(64),
            _fullspec((1, 64)), _fullspec((1, 64)),
            _fullspec((64, 64)), _fullspec((64, 64)),
        ],
        out_specs=[_rowspec(64), _rowspec(64)],
        out_shape=[
            jax.ShapeDtypeStruct((NN, 64), jnp.float32),
            jax.ShapeDtypeStruct((NN, 64), jnp.float32),
        ],
    )(s2, invc, xr2, g2r, be2r, w3rel_t, w3root_t)


def _tc3_body(s3_ref, invc_ref, hr3_ref, g3_ref, be3_ref, res_ref,
              w4rel_ref, w4root_ref, y4_ref, f4r_ref):
    h = s3_ref[...] * invc_ref[...] + hr3_ref[...]
    h = _layer_norm_relu(h, g3_ref[...], be3_ref[...])
    feats = jnp.maximum(h + res_ref[...], 0.0)
    y4 = jnp.dot(feats, w4rel_ref[...], preferred_element_type=jnp.float32)
    y4_ref[...] = jnp.concatenate(
        [y4, jnp.zeros((y4.shape[0], 5), jnp.float32)], axis=-1)
    f4r_ref[...] = jnp.dot(feats, w4root_ref[...], preferred_element_type=jnp.float32)


def _tc3(s3, invc, hr3, g3r, be3r, res, w4rel_t, w4root_t):
    return pl.pallas_call(
        _tc3_body,
        grid=(GRID,),
        in_specs=[
            _rowspec(64), _rowspec(1), _rowspec(64),
            _fullspec((1, 64)), _fullspec((1, 64)), _rowspec(64),
            _fullspec((64, 3)), _fullspec((64, 3)),
        ],
        out_specs=[_rowspec(8), _rowspec(3)],
        out_shape=[
            jax.ShapeDtypeStruct((NN, 8), jnp.float32),
            jax.ShapeDtypeStruct((NN, 3), jnp.float32),
        ],
    )(s3, invc, hr3, g3r, be3r, res, w4rel_t, w4root_t)


def _tc4_body(a0_ref, a1_ref, invc_ref, f4r_ref, out_ref):
    s4 = a0_ref[...] + a1_ref[...]
    out_ref[...] = s4[:, 0:3] * invc_ref[...] + f4r_ref[...]


def _tc4(a0, a1, invc, f4r):
    return pl.pallas_call(
        _tc4_body,
        grid=(GRID,),
        in_specs=[_rowspec(8), _rowspec(8), _rowspec(1), _rowspec(3)],
        out_specs=_rowspec(3),
        out_shape=jax.ShapeDtypeStruct((NN, 3), jnp.float32),
    )(a0, a1, invc, f4r)


def _make_seg_pass(w, split_edges):
    per_tile = ECH // 32 if split_edges else ECH // 16
    n_slots = per_tile // SS
    n_pairs = n_slots // 2

    @functools.partial(
        pl.kernel, mesh=_MESH,
        compiler_params=pltpu.CompilerParams(use_tc_tiling_on_sc=False),
        out_type=jax.ShapeDtypeStruct((2 * ACC_R, w), jnp.float32),
        scratch_types=[
            pltpu.VMEM((2, SS, CH), jnp.int32),
            pltpu.VMEM((2, SS, CH), jnp.int32),
            pltpu.VMEM((2, SS * CH, w), jnp.float32),
            pltpu.VMEM_SHARED((ACC_R, w), jnp.float32),
            pltpu.SemaphoreType.DMA,
            pltpu.SemaphoreType.DMA,
            pltpu.SemaphoreType.DMA,
            pltpu.SemaphoreType.DMA,
        ],
    )
    def seg(ytab_h, src_h, dst_h, zeros_h, out_h,
            sidx, didx, grows, acc, sg0, sg1, ss0, ss1):
        c = lax.axis_index("c")
        s = lax.axis_index("s")
        sg = (sg0, sg1)
        sc = (ss0, ss1)
        rows_per_tile = ACC_R // 16
        pltpu.sync_copy(zeros_h, acc.at[pl.ds(s * rows_per_tile, rows_per_tile)])
        plsc.subcore_barrier()

        if split_edges:
            chunk0 = c * (ECH // 2) + s * per_tile
        else:
            chunk0 = s * per_tile

        def wait_scatters(b):
            for j in range(SS):
                pltpu.make_async_copy(grows.at[b, pl.ds(j * CH, CH)],
                                      acc.at[didx.at[b, j]], sc[b]).wait()

        def load_issue(g, b, wait_scat):
            if wait_scat:
                wait_scatters(b)
            base = chunk0 + g * SS
            pltpu.sync_copy(src_h.at[pl.ds(base, SS)], sidx.at[b])
            pltpu.sync_copy(dst_h.at[pl.ds(base, SS)], didx.at[b])
            if not split_edges:
                off = c * NN
                for j in range(SS):
                    for t in range(CH // 16):
                        sl = pl.ds(t * 16, 16)
                        sidx[b, j, sl] = sidx[b, j, sl] + off
            for j in range(SS):
                pltpu.async_copy(ytab_h.at[sidx.at[b, j]],
                                 grows.at[b, pl.ds(j * CH, CH)], sg[b])

        def drain_scatter(b):
            for j in range(SS):
                pltpu.make_async_copy(ytab_h.at[sidx.at[b, j]],
                                      grows.at[b, pl.ds(j * CH, CH)],
                                      sg[b]).wait()
            for j in range(SS):
                pltpu.async_copy(grows.at[b, pl.ds(j * CH, CH)],
                                 acc.at[didx.at[b, j]], sc[b], add=True)

        load_issue(0, 0, False)
        load_issue(1, 1, False)
        drain_scatter(0)

        def pair(i, carry):
            g0 = 2 * i
            load_issue(g0, 0, True)
            drain_scatter(1)
            load_issue(g0 + 1, 1, True)
            drain_scatter(0)
            return carry
        lax.fori_loop(1, n_pairs, pair, 0)
        drain_scatter(1)
        wait_scatters(0)
        wait_scatters(1)
        plsc.subcore_barrier()

        r = s * rows_per_tile
        pltpu.sync_copy(acc.at[pl.ds(r, rows_per_tile)],
                        out_h.at[pl.ds(c * ACC_R + r, rows_per_tile)])

    return seg


_seg8 = _make_seg_pass(8, True)
_seg32 = _make_seg_pass(32, False)


def _seg_sum8_halves(y8, src_p, dst_p, zeros8):
    out = _seg8(y8, src_p, dst_p, zeros8)
    return out[:NN], out[ACC_R:ACC_R + NN]


def _seg_sum64(y64, src_p, dst_p, zeros32):
    ytab = jnp.concatenate([y64[:, :32], y64[:, 32:]], axis=0)
    out = _seg32(ytab, src_p, dst_p, zeros32)
    return jnp.concatenate([out[:NN], out[ACC_R:ACC_R + NN]], axis=1)


_G_PER_TILE = NGP // CH // 32
_G_PAIRS = _G_PER_TILE // SS // 2


@functools.partial(
    pl.kernel, mesh=_MESH,
    compiler_params=pltpu.CompilerParams(use_tc_tiling_on_sc=False),
    out_type=jax.ShapeDtypeStruct((NGP, 8), jnp.float32),
    scratch_types=[
        pltpu.VMEM((2, SS, CH), jnp.int32),
        pltpu.VMEM((2, SS * CH, 8), jnp.float32),
        pltpu.SemaphoreType.DMA,
        pltpu.SemaphoreType.DMA,
        pltpu.SemaphoreType.DMA,
        pltpu.SemaphoreType.DMA,
    ],
)
def _sc_grid(table_h, idx_h, out_h, gidx, grows, sg0, sg1, ss0, ss1):
    wid = lax.axis_index("s") * 2 + lax.axis_index("c")
    chunk0 = wid * _G_PER_TILE
    sg = (sg0, sg1)
    st = (ss0, ss1)

    def wait_store(b):
        pltpu.make_async_copy(grows.at[b],
                              out_h.at[pl.ds(0, SS * CH)], st[b]).wait()

    def load_issue(g, b, wait_st):
        if wait_st:
            wait_store(b)
        base = chunk0 + g * SS
        pltpu.sync_copy(idx_h.at[pl.ds(base, SS)], gidx.at[b])
        for j in range(SS):
            pltpu.async_copy(table_h.at[gidx.at[b, j]],
                             grows.at[b, pl.ds(j * CH, CH)], sg[b])

    def drain_store(g, b):
        for j in range(SS):
            pltpu.make_async_copy(table_h.at[gidx.at[b, j]],
                                  grows.at[b, pl.ds(j * CH, CH)], sg[b]).wait()
        pltpu.async_copy(grows.at[b],
                         out_h.at[pl.ds((chunk0 + g * SS) * CH, SS * CH)], st[b])

    load_issue(0, 0, False)
    load_issue(1, 1, False)
    drain_store(0, 0)

    def pair(i, carry):
        g0 = 2 * i
        load_issue(g0, 0, True)
        drain_store(g0 - 1, 1)
        load_issue(g0 + 1, 1, True)
        drain_store(g0, 0)
        return carry
    lax.fori_loop(1, _G_PAIRS, pair, 0)
    drain_store(2 * _G_PAIRS - 1, 1)
    wait_store(0)
    wait_store(1)


def _grid_gather(table, idx24):
    idx_pad = jnp.concatenate(
        [idx24.reshape(-1), jnp.zeros((NGP - NGF,), jnp.int32)])
    out = _sc_grid(table, idx_pad.reshape(NGP // CH, CH))
    return out[:NGF].reshape(NN, 192)


def kernel(vertices, edges, vertex_normals, voxel_features, vector_potential,
           w1_root, w1_rel, b1, w2_root, w2_rel, g2, be2, w3_root, w3_rel,
           g3, be3, w_lin, b_lin, w4_root, w4_rel):
    f32 = jnp.float32
    src = jnp.asarray(edges[:, 0], jnp.int32)
    dst = jnp.asarray(edges[:, 1], jnp.int32)
    src_p = jnp.concatenate(
        [src, jnp.zeros((EP - EE,), jnp.int32)]).reshape(ECH, CH)
    dst_p = jnp.concatenate(
        [dst, jnp.full((EP - EE,), NN, jnp.int32)]).reshape(ECH, CH)
    zeros8 = jnp.zeros((ACC_R // 16, 8), f32)
    zeros32 = jnp.zeros((ACC_R // 16, 32), f32)

    table = jnp.transpose(voxel_features, (1, 2, 3, 0)).reshape(64 * 64 * 64, 8)

    idx24, w24 = _tc_pre(vertices, vertex_normals)
    g192 = _grid_gather(table, idx24)

    ones = jnp.ones((NN, 1), f32)
    zeros = jnp.zeros((NN, 1), f32)
    y1 = jnp.concatenate([vertices, vector_potential, ones, zeros], axis=-1)
    a0, a1 = _seg_sum8_halves(y1, src_p, dst_p, zeros8)

    invc, res, xr2, y2 = _tc1(
        a0, a1, g192, w24, vertices, vector_potential,
        w1_rel.T, w1_root.T, b1.reshape(1, 64), w_lin.T, b_lin.reshape(1, 64),
        w2_root.T, w2_rel.T)

    s2 = _seg_sum64(y2, src_p, dst_p, zeros32)
    y3, hr3 = _tc2(s2, invc, xr2, g2.reshape(1, 64), be2.reshape(1, 64),
                   w3_rel.T, w3_root.T)
    s3 = _seg_sum64(y3, src_p, dst_p, zeros32)
    y4, f4r = _tc3(s3, invc, hr3, g3.reshape(1, 64), be3.reshape(1, 64),
                   res, w4_rel.T, w4_root.T)
    b0, b1h = _seg_sum8_halves(y4, src_p, dst_p, zeros8)
    return _tc4(b0, b1h, invc, f4r)


# speedup vs baseline: 4.8664x; 1.1446x over previous
import functools

import jax
import jax.numpy as jnp
from jax import lax
from jax.experimental import pallas as pl
from jax.experimental.pallas import tpu as pltpu
from jax.experimental.pallas import tpu_sc as plsc

NN = 50000
EE = 800000
RB = 2000
GRID = NN // RB

CH = 128
SS = 2
EP = 819200
ECH = EP // CH
ACC_R = 50176
NGF = NN * 24
NGP = 1212416
_sc_cache = {}


def _mesh():
    return plsc.VectorSubcoreMesh(core_axis_name="c", subcore_axis_name="s")


def _rowspec(w):
    return pl.BlockSpec((RB, w), lambda i: (i, 0))


def _fullspec(shape):
    nd = len(shape)
    return pl.BlockSpec(shape, lambda i: (0,) * nd)


def _tc_pre_body(v_ref, n_ref, idx_ref, w_ref):
    v = v_ref[...]
    nrm = n_ref[...]
    r = v.shape[0]
    k8 = lax.broadcasted_iota(jnp.int32, (r, 8), 1)
    dbit = k8 // 4
    hbit = (k8 // 2) % 2
    wbit = k8 % 2
    idx_parts = []
    w_parts = []
    for pt in (v, v + nrm, v - nrm):
        gd = jnp.clip(pt[:, 0:1], 0.0, 63.0)
        gh = jnp.clip(pt[:, 1:2], 0.0, 63.0)
        gw = jnp.clip(pt[:, 2:3], 0.0, 63.0)
        d0 = jnp.floor(gd)
        h0 = jnp.floor(gh)
        w0 = jnp.floor(gw)
        wd = gd - d0
        wh = gh - h0
        ww = gw - w0
        d0i = d0.astype(jnp.int32)
        h0i = h0.astype(jnp.int32)
        w0i = w0.astype(jnp.int32)
        d1i = jnp.minimum(d0i + 1, 63)
        h1i = jnp.minimum(h0i + 1, 63)
        w1i = jnp.minimum(w0i + 1, 63)
        dsel = jnp.where(dbit == 1, d1i, d0i)
        hsel = jnp.where(hbit == 1, h1i, h0i)
        wsel = jnp.where(wbit == 1, w1i, w0i)
        idx_parts.append((dsel * 64 + hsel) * 64 + wsel)
        wdp = jnp.where(dbit == 1, wd, 1.0 - wd)
        whp = jnp.where(hbit == 1, wh, 1.0 - wh)
        wwp = jnp.where(wbit == 1, ww, 1.0 - ww)
        w_parts.append(wdp * whp * wwp)
    idx_ref[...] = jnp.concatenate(idx_parts, axis=-1)
    w_ref[...] = jnp.concatenate(w_parts, axis=-1)


def _tc_pre(vertices, vertex_normals):
    return pl.pallas_call(
        _tc_pre_body,
        grid=(GRID,),
        in_specs=[_rowspec(3), _rowspec(3)],
        out_specs=[_rowspec(24), _rowspec(24)],
        out_shape=[
            jax.ShapeDtypeStruct((NN, 24), jnp.int32),
            jax.ShapeDtypeStruct((NN, 24), jnp.float32),
        ],
    )(vertices, vertex_normals)


def _tc1_body(a0_ref, a1_ref, g_ref, w24_ref, v_ref, vp_ref,
              w1rel_ref, w1root_ref, b1_ref, r24_ref,
              wl_top_ref, wl_mid_ref, wl_bot_ref, blin_ref,
              wr_top_ref, wr_mid_ref, wr_bot_ref,
              wy_top_ref, wy_mid_ref, wy_bot_ref,
              invc_ref, res_ref, xr2_ref, y2_ref):
    def dot(a, b):
        return jnp.dot(a, b, preferred_element_type=jnp.float32)
    s1 = a0_ref[...] + a1_ref[...]
    cnt = s1[:, 6:7]
    invc = 1.0 / jnp.maximum(cnt, 1.0)
    invc_ref[...] = invc
    mean1 = s1[:, 0:6] * invc
    inpf = jnp.concatenate([v_ref[...], vp_ref[...]], axis=-1)
    feats = dot(mean1, w1rel_ref[...]) + dot(inpf, w1root_ref[...]) + b1_ref[...]
    prod = g_ref[...] * dot(w24_ref[...], r24_ref[...])
    res_ref[...] = (dot(feats, wl_top_ref[...]) + dot(prod, wl_mid_ref[...])
                    + dot(inpf, wl_bot_ref[...]) + blin_ref[...])
    xr2_ref[...] = (dot(feats, wr_top_ref[...]) + dot(prod, wr_mid_ref[...])
                    + dot(inpf, wr_bot_ref[...]))
    y2_ref[...] = (dot(feats, wy_top_ref[...]) + dot(prod, wy_mid_ref[...])
                   + dot(inpf, wy_bot_ref[...]))


def _tc1(a0, a1, g192, w24, vertices, vp, w1rel_t, w1root_t, b1r, r24,
         wl, blinr, wr, wy):
    return pl.pallas_call(
        _tc1_body,
        grid=(GRID,),
        in_specs=[
            _rowspec(8), _rowspec(8), _rowspec(192), _rowspec(24),
            _rowspec(3), _rowspec(3),
            _fullspec((6, 64)), _fullspec((6, 64)), _fullspec((1, 64)),
            _fullspec((24, 192)),
            _fullspec((64, 64)), _fullspec((192, 64)), _fullspec((6, 64)),
            _fullspec((1, 64)),
            _fullspec((64, 64)), _fullspec((192, 64)), _fullspec((6, 64)),
            _fullspec((64, 64)), _fullspec((192, 64)), _fullspec((6, 64)),
        ],
        out_specs=[_rowspec(1), _rowspec(64), _rowspec(64), _rowspec(64)],
        out_shape=[
            jax.ShapeDtypeStruct((NN, 1), jnp.float32),
            jax.ShapeDtypeStruct((NN, 64), jnp.float32),
            jax.ShapeDtypeStruct((NN, 64), jnp.float32),
            jax.ShapeDtypeStruct((NN, 64), jnp.float32),
        ],
    )(a0, a1, g192, w24, vertices, vp, w1rel_t, w1root_t, b1r, r24,
      wl[0], wl[1], wl[2], blinr, wr[0], wr[1], wr[2], wy[0], wy[1], wy[2])


def _layer_norm_relu(h, g, b):
    mu = jnp.mean(h, axis=-1, keepdims=True)
    var = jnp.mean((h - mu) * (h - mu), axis=-1, keepdims=True)
    hn = (h - mu) * jax.lax.rsqrt(var + 1e-5) * g + b
    return jnp.maximum(hn, 0.0)


def _tc2_body(s2_ref, invc_ref, xr2_ref, g2_ref, be2_ref,
              w3rel_ref, w3root_ref, y3_ref, hr3_ref):
    h = s2_ref[...] * invc_ref[...] + xr2_ref[...]
    h = _layer_norm_relu(h, g2_ref[...], be2_ref[...])
    y3_ref[...] = jnp.dot(h, w3rel_ref[...], preferred_element_type=jnp.float32)
    hr3_ref[...] = jnp.dot(h, w3root_ref[...], preferred_element_type=jnp.float32)


def _tc2(s2, invc, xr2, g2r, be2r, w3rel_t, w3root_t):
    return pl.pallas_call(
        _tc2_body,
        grid=(GRID,),
        in_specs=[
            _rowspec(64), _rowspec(1), _rowspec(64),
            _fullspec((1, 64)), _fullspec((1, 64)),
            _fullspec((64, 64)), _fullspec((64, 64)),
        ],
        out_specs=[_rowspec(64), _rowspec(64)],
        out_shape=[
            jax.ShapeDtypeStruct((NN, 64), jnp.float32),
            jax.ShapeDtypeStruct((NN, 64), jnp.float32),
        ],
    )(s2, invc, xr2, g2r, be2r, w3rel_t, w3root_t)


def _tc3_body(s3_ref, invc_ref, hr3_ref, g3_ref, be3_ref, res_ref,
              w4rel_ref, w4root_ref, y4_ref, f4r_ref):
    h = s3_ref[...] * invc_ref[...] + hr3_ref[...]
    h = _layer_norm_relu(h, g3_ref[...], be3_ref[...])
    feats = jnp.maximum(h + res_ref[...], 0.0)
    y4 = jnp.dot(feats, w4rel_ref[...], preferred_element_type=jnp.float32)
    y4_ref[...] = jnp.concatenate(
        [y4, jnp.zeros((y4.shape[0], 5), jnp.float32)], axis=-1)
    f4r_ref[...] = jnp.dot(feats, w4root_ref[...], preferred_element_type=jnp.float32)


def _tc3(s3, invc, hr3, g3r, be3r, res, w4rel_t, w4root_t):
    return pl.pallas_call(
        _tc3_body,
        grid=(GRID,),
        in_specs=[
            _rowspec(64), _rowspec(1), _rowspec(64),
            _fullspec((1, 64)), _fullspec((1, 64)), _rowspec(64),
            _fullspec((64, 3)), _fullspec((64, 3)),
        ],
        out_specs=[_rowspec(8), _rowspec(3)],
        out_shape=[
            jax.ShapeDtypeStruct((NN, 8), jnp.float32),
            jax.ShapeDtypeStruct((NN, 3), jnp.float32),
        ],
    )(s3, invc, hr3, g3r, be3r, res, w4rel_t, w4root_t)


def _tc4_body(a0_ref, a1_ref, invc_ref, f4r_ref, out_ref):
    s4 = a0_ref[...] + a1_ref[...]
    out_ref[...] = s4[:, 0:3] * invc_ref[...] + f4r_ref[...]


def _tc4(a0, a1, invc, f4r):
    return pl.pallas_call(
        _tc4_body,
        grid=(GRID,),
        in_specs=[_rowspec(8), _rowspec(8), _rowspec(1), _rowspec(3)],
        out_specs=_rowspec(3),
        out_shape=jax.ShapeDtypeStruct((NN, 3), jnp.float32),
    )(a0, a1, invc, f4r)


def _make_seg_pass(w, split_edges):
    key = ("seg", w, split_edges)
    if key in _sc_cache:
        return _sc_cache[key]
    per_tile = ECH // 32 if split_edges else ECH // 16
    n_slots = per_tile // SS
    n_pairs = n_slots // 2

    @functools.partial(
        pl.kernel, mesh=_mesh(),
        compiler_params=pltpu.CompilerParams(use_tc_tiling_on_sc=False),
        out_type=jax.ShapeDtypeStruct((2 * ACC_R, w), jnp.float32),
        scratch_types=[
            pltpu.VMEM((2, SS, CH), jnp.int32),
            pltpu.VMEM((2, SS, CH), jnp.int32),
            pltpu.VMEM((2, SS * CH, w), jnp.float32),
            pltpu.VMEM_SHARED((ACC_R, w), jnp.float32),
            pltpu.SemaphoreType.DMA,
            pltpu.SemaphoreType.DMA,
            pltpu.SemaphoreType.DMA,
            pltpu.SemaphoreType.DMA,
        ],
    )
    def seg(ytab_h, src_h, dst_h, zeros_h, out_h,
            sidx, didx, grows, acc, sg0, sg1, ss0, ss1):
        c = lax.axis_index("c")
        s = lax.axis_index("s")
        sg = (sg0, sg1)
        sc = (ss0, ss1)
        rows_per_tile = ACC_R // 16
        pltpu.sync_copy(zeros_h, acc.at[pl.ds(s * rows_per_tile, rows_per_tile)])
        plsc.subcore_barrier()

        if split_edges:
            chunk0 = c * (ECH // 2) + s * per_tile
        else:
            chunk0 = s * per_tile

        def wait_scatters(b):
            for j in range(SS):
                pltpu.make_async_copy(grows.at[b, pl.ds(j * CH, CH)],
                                      acc.at[didx.at[b, j]], sc[b]).wait()

        def load_issue(g, b, wait_scat):
            if wait_scat:
                wait_scatters(b)
            base = chunk0 + g * SS
            pltpu.sync_copy(src_h.at[pl.ds(base, SS)], sidx.at[b])
            pltpu.sync_copy(dst_h.at[pl.ds(base, SS)], didx.at[b])
            if not split_edges:
                off = c * NN
                for j in range(SS):
                    for t in range(CH // 16):
                        sl = pl.ds(t * 16, 16)
                        sidx[b, j, sl] = sidx[b, j, sl] + off
            for j in range(SS):
                pltpu.async_copy(ytab_h.at[sidx.at[b, j]],
                                 grows.at[b, pl.ds(j * CH, CH)], sg[b])

        def drain_scatter(b):
            for j in range(SS):
                pltpu.make_async_copy(ytab_h.at[sidx.at[b, j]],
                                      grows.at[b, pl.ds(j * CH, CH)],
                                      sg[b]).wait()
            for j in range(SS):
                pltpu.async_copy(grows.at[b, pl.ds(j * CH, CH)],
                                 acc.at[didx.at[b, j]], sc[b], add=True)

        load_issue(0, 0, False)
        load_issue(1, 1, False)
        drain_scatter(0)

        def pair(i, carry):
            g0 = 2 * i
            load_issue(g0, 0, True)
            drain_scatter(1)
            load_issue(g0 + 1, 1, True)
            drain_scatter(0)
            return carry
        lax.fori_loop(1, n_pairs, pair, 0)
        drain_scatter(1)
        wait_scatters(0)
        wait_scatters(1)
        plsc.subcore_barrier()

        r = s * rows_per_tile
        pltpu.sync_copy(acc.at[pl.ds(r, rows_per_tile)],
                        out_h.at[pl.ds(c * ACC_R + r, rows_per_tile)])

    _sc_cache[key] = seg
    return seg


def _seg_sum8_halves(y8, src_p, dst_p, zeros8):
    out = _make_seg_pass(8, True)(y8, src_p, dst_p, zeros8)
    return out[:NN], out[ACC_R:ACC_R + NN]


def _seg_sum64(y64, src_p, dst_p, zeros32):
    ytab = jnp.concatenate([y64[:, :32], y64[:, 32:]], axis=0)
    out = _make_seg_pass(32, False)(ytab, src_p, dst_p, zeros32)
    return jnp.concatenate([out[:NN], out[ACC_R:ACC_R + NN]], axis=1)


_G_PER_TILE = NGP // CH // 32
_G_PAIRS = _G_PER_TILE // SS // 2


def _make_grid_kernel():
    key = ("grid",)
    if key in _sc_cache:
        return _sc_cache[key]

    @functools.partial(
        pl.kernel, mesh=_mesh(),
        compiler_params=pltpu.CompilerParams(use_tc_tiling_on_sc=False),
        out_type=jax.ShapeDtypeStruct((NGP, 8), jnp.float32),
        scratch_types=[
            pltpu.VMEM((2, SS, CH), jnp.int32),
            pltpu.VMEM((2, SS * CH, 8), jnp.float32),
            pltpu.SemaphoreType.DMA,
            pltpu.SemaphoreType.DMA,
            pltpu.SemaphoreType.DMA,
            pltpu.SemaphoreType.DMA,
        ],
    )
    def _sc_grid(table_h, idx_h, out_h, gidx, grows, sg0, sg1, ss0, ss1):
        wid = lax.axis_index("s") * 2 + lax.axis_index("c")
        chunk0 = wid * _G_PER_TILE
        sg = (sg0, sg1)
        st = (ss0, ss1)

        def wait_store(b):
            pltpu.make_async_copy(grows.at[b],
                                  out_h.at[pl.ds(0, SS * CH)], st[b]).wait()

        def load_issue(g, b, wait_st):
            if wait_st:
                wait_store(b)
            base = chunk0 + g * SS
            pltpu.sync_copy(idx_h.at[pl.ds(base, SS)], gidx.at[b])
            for j in range(SS):
                pltpu.async_copy(table_h.at[gidx.at[b, j]],
                                 grows.at[b, pl.ds(j * CH, CH)], sg[b])

        def drain_store(g, b):
            for j in range(SS):
                pltpu.make_async_copy(table_h.at[gidx.at[b, j]],
                                      grows.at[b, pl.ds(j * CH, CH)],
                                      sg[b]).wait()
            pltpu.async_copy(grows.at[b],
                             out_h.at[pl.ds((chunk0 + g * SS) * CH, SS * CH)],
                             st[b])

        load_issue(0, 0, False)
        load_issue(1, 1, False)
        drain_store(0, 0)

        def pair(i, carry):
            g0 = 2 * i
            load_issue(g0, 0, True)
            drain_store(g0 - 1, 1)
            load_issue(g0 + 1, 1, True)
            drain_store(g0, 0)
            return carry
        lax.fori_loop(1, _G_PAIRS, pair, 0)
        drain_store(2 * _G_PAIRS - 1, 1)
        wait_store(0)
        wait_store(1)

    _sc_cache[key] = _sc_grid
    return _sc_grid


def _grid_gather(table, idx24):
    idx_pad = jnp.concatenate(
        [idx24.reshape(-1), jnp.zeros((NGP - NGF,), jnp.int32)])
    out = _make_grid_kernel()(table, idx_pad.reshape(NGP // CH, CH))
    return out[:NGF].reshape(NN, 192)


def kernel(vertices, edges, vertex_normals, voxel_features, vector_potential,
           w1_root, w1_rel, b1, w2_root, w2_rel, g2, be2, w3_root, w3_rel,
           g3, be3, w_lin, b_lin, w4_root, w4_rel):
    f32 = jnp.float32
    src = jnp.asarray(edges[:, 0], jnp.int32)
    dst = jnp.asarray(edges[:, 1], jnp.int32)
    src_p = jnp.concatenate(
        [src, jnp.zeros((EP - EE,), jnp.int32)]).reshape(ECH, CH)
    dst_p = jnp.concatenate(
        [dst, jnp.full((EP - EE,), NN, jnp.int32)]).reshape(ECH, CH)
    zeros8 = jnp.zeros((ACC_R // 16, 8), f32)
    zeros32 = jnp.zeros((ACC_R // 16, 32), f32)

    table = jnp.transpose(voxel_features, (1, 2, 3, 0)).reshape(64 * 64 * 64, 8)

    idx24, w24 = _tc_pre(vertices, vertex_normals)
    g192 = _grid_gather(table, idx24)

    ones = jnp.ones((NN, 1), f32)
    zeros = jnp.zeros((NN, 1), f32)
    y1 = jnp.concatenate([vertices, vector_potential, ones, zeros], axis=-1)
    a0, a1 = _seg_sum8_halves(y1, src_p, dst_p, zeros8)

    j24 = jnp.arange(24, dtype=jnp.int32)
    m192 = jnp.arange(192, dtype=jnp.int32)
    r24 = (j24[:, None] == (m192[None, :] // 8)).astype(f32)
    pcol = (m192 // 64) * 8 + (m192 % 8)
    s192 = (pcol[:, None] == j24[None, :]).astype(f32)

    def split3(wt):
        return (wt[:64], s192 @ wt[64:88], wt[88:])
    invc, res, xr2, y2 = _tc1(
        a0, a1, g192, w24, vertices, vector_potential,
        w1_rel.T, w1_root.T, b1.reshape(1, 64), r24,
        split3(w_lin.T), b_lin.reshape(1, 64),
        split3(w2_root.T), split3(w2_rel.T))

    s2 = _seg_sum64(y2, src_p, dst_p, zeros32)
    y3, hr3 = _tc2(s2, invc, xr2, g2.reshape(1, 64), be2.reshape(1, 64),
                   w3_rel.T, w3_root.T)
    s3 = _seg_sum64(y3, src_p, dst_p, zeros32)
    y4, f4r = _tc3(s3, invc, hr3, g3.reshape(1, 64), be3.reshape(1, 64),
                   res, w4_rel.T, w4_root.T)
    b0, b1h = _seg_sum8_halves(y4, src_p, dst_p, zeros8)
    return _tc4(b0, b1h, invc, f4r)
